# baseline (speedup 1.0000x reference)
import functools

import jax
import jax.numpy as jnp
from jax import lax
from jax.experimental import pallas as pl
from jax.experimental.pallas import tpu as pltpu
from jax.experimental.pallas import tpu_sc as plsc

_S, _B, _D = 2048, 4, 1024
_L = 16
_NC, _NS = 2, 16
_NW = _NC * _NS
_RPW = _S // _NW
_R = 8
_NCH = _RPW // _R
_NBUF = 3


def _sc_body(x_hbm, pos_hbm, out_hbm, xbuf, pbuf, *sems):
    wid = lax.axis_index("s") * _NC + lax.axis_index("c")
    rbase = wid * _RPW
    sx = sems[0:_NBUF]
    sp = sems[_NBUF:2 * _NBUF]
    so = sems[2 * _NBUF:3 * _NBUF]
    loads = [None] * _NCH
    stores = [None] * _NCH

    def start_load(g):
        b = g % _NBUF
        row0 = rbase + g * _R
        cs = []
        for t in range(_D // 128):
            cs.append(pltpu.async_copy(
                x_hbm.at[pl.ds(row0, _R), :, pl.ds(t * 128, 128)],
                xbuf.at[b, :, :, pl.ds(t * 128, 128)], sx[b]))
            cs.append(pltpu.async_copy(
                pos_hbm.at[pl.ds(row0, _R), pl.ds(t * 128, 128)],
                pbuf.at[b, :, pl.ds(t * 128, 128)], sp[b]))
        loads[g] = cs

    for g in range(min(_NBUF - 1, _NCH)):
        start_load(g)
    for g in range(_NCH):
        b = g % _NBUF
        for c in loads[g]:
            c.wait()

        @plsc.parallel_loop(0, _R * _D, step=_L, unroll=8)
        def _accumulate(q, _b=b):
            q = pl.multiple_of(q, _L)
            i = q >> 10
            j = pl.multiple_of(q & (_D - 1), _L)
            pvec = pbuf[_b, i, pl.ds(j, _L)]
            for bb in range(_B):
                plsc.addupdate(xbuf.at[_b, i, bb, pl.ds(j, _L)], pvec)

        stores[g] = [
            pltpu.async_copy(
                xbuf.at[b, :, :, pl.ds(t * 128, 128)],
                out_hbm.at[pl.ds(rbase + g * _R, _R), :, pl.ds(t * 128, 128)],
                so[b])
            for t in range(_D // 128)]
        nxt = g + _NBUF - 1
        if nxt < _NCH:
            prev = nxt - _NBUF
            if prev >= 0:
                for c in stores[prev]:
                    c.wait()
            start_load(nxt)
    for g in range(max(0, _NCH - _NBUF), _NCH):
        if stores[g] is not None:
            for c in stores[g]:
                c.wait()


@jax.jit
def _sc_add(x, pos_table):
    run = pl.kernel(
        _sc_body,
        out_type=jax.ShapeDtypeStruct((_S, _B, _D), jnp.float32),
        mesh=plsc.VectorSubcoreMesh(
            core_axis_name="c", subcore_axis_name="s",
            num_cores=_NC, num_subcores=_NS),
        scratch_types=(
            [pltpu.VMEM((_NBUF, _R, _B, _D), jnp.float32),
             pltpu.VMEM((_NBUF, _R, _D), jnp.float32)]
            + [pltpu.SemaphoreType.DMA] * (3 * _NBUF)
        ),
        compiler_params=pltpu.CompilerParams(use_tc_tiling_on_sc=True),
    )
    return run(x, pos_table)


def kernel(x, pos_table):
    return _sc_add(x, pos_table[: x.shape[0]])


# speedup vs baseline: 1.1030x; 1.1030x over previous
import jax
import jax.numpy as jnp
from jax import lax
from jax.experimental import pallas as pl
from jax.experimental.pallas import tpu as pltpu
from jax.experimental.pallas import tpu_sc as plsc

_S, _B, _D = 2048, 4, 1024
_L = 16
_NC, _NS = 2, 16
_NW = _NC * _NS
_RPW = _S // _NW
_R = 8
_NCH = _RPW // _R
_PC = _R * _D


def _sc_body(x_hbm, pos_hbm, out_hbm, xbuf, pbuf, sx0, sx1, sp0, sp1, so0, so1):
    xf = x_hbm.reshape(_S * _B, _D)
    pf = pos_hbm
    of = out_hbm.reshape(_S * _B, _D)
    wid = lax.axis_index("s") * _NC + lax.axis_index("c")
    xbase = wid * _RPW
    pbase = wid * _RPW
    sx = (sx0, sx1)
    sp = (sp0, sp1)
    so = (so0, so1)
    loads = [None, None]
    stores = [None, None]

    def start_load(g):
        b = g % 2
        cx = pltpu.async_copy(
            xf.at[pl.ds((xbase + g * _R) * _B, _R * _B)], xbuf.at[b], sx[b])
        cp = pltpu.async_copy(
            pf.at[pl.ds(pbase + g * _R, _R)], pbuf.at[b], sp[b])
        loads[b] = (cx, cp)

    start_load(0)
    for g in range(_NCH):
        b = g % 2
        if g + 1 < _NCH:
            if stores[1 - b] is not None:
                stores[1 - b].wait()
            start_load(g + 1)
        cx, cp = loads[b]
        cx.wait()
        cp.wait()

        @plsc.parallel_loop(0, _PC, step=_L, unroll=8)
        def _accumulate(q, _b=b):
            q = pl.multiple_of(q, _L)
            i = q >> 10
            j = pl.multiple_of(q & (_D - 1), _L)
            pvec = pbuf[_b, i, pl.ds(j, _L)]
            i4 = i << 2
            for bb in range(_B):
                plsc.addupdate(xbuf.at[_b, i4 + bb, pl.ds(j, _L)], pvec)

        stores[b] = pltpu.async_copy(
            xbuf.at[b], of.at[pl.ds((xbase + g * _R) * _B, _R * _B)], so[b])
    stores[0].wait()
    stores[1].wait()


@jax.jit
def _sc_add(x, pos_table):
    run = pl.kernel(
        _sc_body,
        out_type=jax.ShapeDtypeStruct((_S, _B, _D), jnp.float32),
        mesh=plsc.VectorSubcoreMesh(
            core_axis_name="c", subcore_axis_name="s",
            num_cores=_NC, num_subcores=_NS),
        scratch_types=[
            pltpu.VMEM((2, _R * _B, _D), jnp.float32),
            pltpu.VMEM((2, _R, _D), jnp.float32),
            pltpu.SemaphoreType.DMA,
            pltpu.SemaphoreType.DMA,
            pltpu.SemaphoreType.DMA,
            pltpu.SemaphoreType.DMA,
            pltpu.SemaphoreType.DMA,
            pltpu.SemaphoreType.DMA,
        ],
        compiler_params=pltpu.CompilerParams(use_tc_tiling_on_sc=True),
    )
    return run(x, pos_table)


def kernel(x, pos_table):
    return _sc_add(x, pos_table[: x.shape[0]])


# speedup vs baseline: 1.1146x; 1.0106x over previous
import jax
import jax.numpy as jnp
from jax import lax
from jax.experimental import pallas as pl
from jax.experimental.pallas import tpu as pltpu
from jax.experimental.pallas import tpu_sc as plsc

_S, _B, _D = 2048, 4, 1024
_L = 16
_NC, _NS = 2, 16
_NW = _NC * _NS
_RPW = _S // _NW
_R = 8
_NCH = _RPW // _R
_PC = _R * _D


def _sc_body(x_hbm, pos_hbm, out_hbm, xbuf, pbuf, sx0, sx1, sp0, sp1, so0, so1):
    xf = x_hbm.reshape(_S * _B, _D)
    pf = pos_hbm
    of = out_hbm.reshape(_S * _B, _D)
    wid = lax.axis_index("s") * _NC + lax.axis_index("c")
    xbase = wid * _RPW
    pbase = wid * _RPW
    sx = (sx0, sx1)
    sp = (sp0, sp1)
    so = (so0, so1)
    loads = [None, None]
    stores = [None, None]

    def start_load(g):
        b = g % 2
        cx = pltpu.async_copy(
            xf.at[pl.ds((xbase + g * _R) * _B, _R * _B)], xbuf.at[b], sx[b])
        cp = pltpu.async_copy(
            pf.at[pl.ds(pbase + g * _R, _R)], pbuf.at[b], sp[b])
        loads[b] = (cx, cp)

    start_load(0)
    for g in range(_NCH):
        b = g % 2
        if g + 1 < _NCH:
            if stores[1 - b] is not None:
                stores[1 - b].wait()
            start_load(g + 1)
        cx, cp = loads[b]
        cx.wait()
        cp.wait()

        @plsc.parallel_loop(0, _PC, step=_L, unroll=4)
        def _accumulate(q, _b=b):
            q = pl.multiple_of(q, _L)
            i = q >> 10
            j = pl.multiple_of(q & (_D - 1), _L)
            pvec = pbuf[_b, i, pl.ds(j, _L)]
            i4 = i << 2
            for bb in range(_B):
                plsc.addupdate(xbuf.at[_b, i4 + bb, pl.ds(j, _L)], pvec)

        stores[b] = pltpu.async_copy(
            xbuf.at[b], of.at[pl.ds((xbase + g * _R) * _B, _R * _B)], so[b])
    stores[0].wait()
    stores[1].wait()


@jax.jit
def _sc_add(x, pos_table):
    run = pl.kernel(
        _sc_body,
        out_type=jax.ShapeDtypeStruct((_S, _B, _D), jnp.float32),
        mesh=plsc.VectorSubcoreMesh(
            core_axis_name="c", subcore_axis_name="s",
            num_cores=_NC, num_subcores=_NS),
        scratch_types=[
            pltpu.VMEM((2, _R * _B, _D), jnp.float32),
            pltpu.VMEM((2, _R, _D), jnp.float32),
            pltpu.SemaphoreType.DMA,
            pltpu.SemaphoreType.DMA,
            pltpu.SemaphoreType.DMA,
            pltpu.SemaphoreType.DMA,
            pltpu.SemaphoreType.DMA,
            pltpu.SemaphoreType.DMA,
        ],
        compiler_params=pltpu.CompilerParams(use_tc_tiling_on_sc=True),
    )
    return run(x, pos_table)


def kernel(x, pos_table):
    return _sc_add(x, pos_table[: x.shape[0]])
